# baseline (speedup 1.0000x reference)
import functools

import jax
import jax.numpy as jnp
from jax import lax
from jax.experimental import pallas as pl
from jax.experimental.pallas import tpu as pltpu
from jax.experimental.pallas import tpu_sc as plsc

N = 10000
E = 320000
D = 128
ORDER = 3

NC = 2
NS = 16
F = D // NC
EPT = E // NS
CH = 256
NCHUNK = 80
SLOTS = NCHUNK * CH
PADN = 16
NBUF = 2
RQ = 624
ZR = 104
LANES = 16


def _sc_agg_body(y_hbm, src_hbm, dst_hbm, out_hbm,
                 src_v, dst_v, rows, zbuf, acc_sh, sems, ssems):
    c = lax.axis_index("c")
    s = lax.axis_index("s")
    base = pl.multiple_of(s * RQ, 8)

    pltpu.sync_copy(src_hbm.at[c, s], src_v)
    pltpu.sync_copy(dst_hbm.at[s], dst_v)

    def zrow(i, carry):
        for j in range(F // LANES):
            zbuf[i, pl.ds(j * LANES, LANES)] = jnp.zeros((LANES,), jnp.float32)
        return carry
    lax.fori_loop(0, ZR, zrow, 0)
    for r in range(RQ // ZR):
        pltpu.sync_copy(zbuf, acc_sh.at[pl.ds(base + r * ZR, ZR)])

    @pl.when(s == NS - 1)
    def _():
        pltpu.sync_copy(zbuf.at[pl.ds(0, N - NS * RQ)],
                        acc_sh.at[pl.ds(NS * RQ, N - NS * RQ)])

    plsc.subcore_barrier()

    for b in range(NBUF):
        pltpu.async_copy(y_hbm.at[src_v.at[b]], rows.at[b], sems.at[b])
    pltpu.make_async_copy(y_hbm.at[src_v.at[0]], rows.at[0],
                          sems.at[0]).wait()
    pltpu.async_copy(rows.at[0], acc_sh.at[dst_v.at[0]],
                     ssems.at[0], add=True)

    @pl.loop(1, NCHUNK - NBUF + 1, step=NBUF)
    def _(k0):
        for j in range(NBUF):
            k = k0 + j
            b = (1 + j) % NBUF
            rb = j % NBUF
            pltpu.make_async_copy(y_hbm.at[src_v.at[0]], rows.at[rb],
                                  ssems.at[rb]).wait()
            pltpu.async_copy(y_hbm.at[src_v.at[k + NBUF - 1]], rows.at[rb],
                             sems.at[rb])
            pltpu.make_async_copy(y_hbm.at[src_v.at[0]], rows.at[b],
                                  sems.at[b]).wait()
            pltpu.async_copy(rows.at[b], acc_sh.at[dst_v.at[k]],
                             ssems.at[b], add=True)

    for t in range(NBUF - 1):
        k = NCHUNK - NBUF + 1 + t
        b = (1 + t) % NBUF
        pltpu.make_async_copy(y_hbm.at[src_v.at[0]], rows.at[b],
                              sems.at[b]).wait()
        pltpu.async_copy(rows.at[b], acc_sh.at[dst_v.at[k]],
                         ssems.at[b], add=True)
    for b in range(NBUF):
        pltpu.make_async_copy(y_hbm.at[src_v.at[0]], rows.at[b],
                              ssems.at[b]).wait()

    plsc.subcore_barrier()

    pltpu.sync_copy(acc_sh.at[pl.ds(base, RQ)],
                    out_hbm.at[c, pl.ds(base, RQ)])

    @pl.when(s == NS - 1)
    def _():
        pltpu.sync_copy(acc_sh.at[pl.ds(NS * RQ, N - NS * RQ)],
                        out_hbm.at[c, pl.ds(NS * RQ, N - NS * RQ)])


def _sc_agg(y2, src2, dst):
    mesh = plsc.VectorSubcoreMesh(core_axis_name="c", subcore_axis_name="s",
                                  num_cores=NC, num_subcores=NS)
    f = pl.kernel(
        _sc_agg_body,
        out_type=jax.ShapeDtypeStruct((NC, N, F), jnp.float32),
        mesh=mesh,
        scratch_types=[
            pltpu.VMEM((NCHUNK, CH), jnp.int32),
            pltpu.VMEM((NCHUNK, CH), jnp.int32),
            pltpu.VMEM((NBUF, CH, F), jnp.float32),
            pltpu.VMEM((ZR, F), jnp.float32),
            pltpu.VMEM_SHARED((N + PADN, F), jnp.float32),
            pltpu.SemaphoreType.DMA((NBUF,)),
            pltpu.SemaphoreType.DMA((NBUF,)),
        ],
        compiler_params=pltpu.CompilerParams(use_tc_tiling_on_sc=False),
    )
    return f(y2, src2, dst)


def _tc_layer_body(p_ref, w_ref, b_ref, c_ref, o_ref, *, poly):
    x = jnp.concatenate([p_ref[0], p_ref[1]], axis=-1)
    h = jnp.dot(x, w_ref[...], preferred_element_type=jnp.float32) + b_ref[...]
    if poly:
        c0 = c_ref[0:1, :]
        c1 = c_ref[1:2, :]
        c2 = c_ref[2:3, :]
        c3 = c_ref[3:4, :]
        h = c0 + h * (c1 + h * (c2 + h * c3))
    o_ref[...] = h


def _tc_layer(partials, W, b, crow, poly):
    BN = 1000
    return pl.pallas_call(
        functools.partial(_tc_layer_body, poly=poly),
        grid=(N // BN,),
        in_specs=[
            pl.BlockSpec((NC, BN, F), lambda i: (0, i, 0)),
            pl.BlockSpec((D, D), lambda i: (0, 0)),
            pl.BlockSpec((1, D), lambda i: (0, 0)),
            pl.BlockSpec((ORDER + 1, D), lambda i: (0, 0)),
        ],
        out_specs=pl.BlockSpec((BN, D), lambda i: (i, 0)),
        out_shape=jax.ShapeDtypeStruct((N, D), jnp.float32),
    )(partials, W, b.reshape(1, D), crow)


def kernel(nf_mat, conv_mat, W1, b1, W2, b2, W3, b3, coeffs):
    pad = SLOTS - EPT
    pad_src = jnp.broadcast_to((jnp.arange(pad, dtype=jnp.int32) * 37) % N,
                               (NS, pad))
    pad_dst = jnp.broadcast_to(N + (jnp.arange(pad, dtype=jnp.int32) % PADN),
                               (NS, pad))
    src = jnp.concatenate([conv_mat[0].reshape(NS, EPT), pad_src], axis=1)
    dst = jnp.concatenate([conv_mat[1].reshape(NS, EPT), pad_dst], axis=1)
    src = src.reshape(NS, NCHUNK, CH)
    dst = dst.reshape(NS, NCHUNK, CH)
    src2 = jnp.stack([src * 2, src * 2 + 1])
    cb = jnp.broadcast_to(coeffs[:, :, None], (2, ORDER + 1, D))

    p = _sc_agg(nf_mat.reshape(NC * N, F), src2, dst)
    h = _tc_layer(p, W1, b1, cb[0], poly=True)
    p = _sc_agg(h.reshape(NC * N, F), src2, dst)
    h = _tc_layer(p, W2, b2, cb[1], poly=True)
    p = _sc_agg(h.reshape(NC * N, F), src2, dst)
    return _tc_layer(p, W3, b3, cb[0], poly=False)


# speedup vs baseline: 1.0630x; 1.0630x over previous
import functools

import jax
import jax.numpy as jnp
from jax import lax
from jax.experimental import pallas as pl
from jax.experimental.pallas import tpu as pltpu
from jax.experimental.pallas import tpu_sc as plsc

N = 10000
E = 320000
D = 128
ORDER = 3

NC = 2
NS = 16
F = D // NC
EPT = E // NS
CH = 160
NCHUNK = 128
SLOTS = NCHUNK * CH
PADN = 16
NBUF = 4
RQ = 624
ZR = 104
LANES = 16


def _sc_agg_body(y_hbm, src_hbm, dst_hbm, out_hbm,
                 src_v, dst_v, rows, zbuf, acc_sh, sems, ssems):
    c = lax.axis_index("c")
    s = lax.axis_index("s")
    base = pl.multiple_of(s * RQ, 8)

    pltpu.sync_copy(src_hbm.at[c, s], src_v)
    pltpu.sync_copy(dst_hbm.at[s], dst_v)

    def zrow(i, carry):
        for j in range(F // LANES):
            zbuf[i, pl.ds(j * LANES, LANES)] = jnp.zeros((LANES,), jnp.float32)
        return carry
    lax.fori_loop(0, ZR, zrow, 0)
    for r in range(RQ // ZR):
        pltpu.sync_copy(zbuf, acc_sh.at[pl.ds(base + r * ZR, ZR)])

    @pl.when(s == NS - 1)
    def _():
        pltpu.sync_copy(zbuf.at[pl.ds(0, N - NS * RQ)],
                        acc_sh.at[pl.ds(NS * RQ, N - NS * RQ)])

    plsc.subcore_barrier()

    for b in range(NBUF):
        pltpu.async_copy(y_hbm.at[src_v.at[b]], rows.at[b], sems.at[b])
    pltpu.make_async_copy(y_hbm.at[src_v.at[0]], rows.at[0],
                          sems.at[0]).wait()
    pltpu.async_copy(rows.at[0], acc_sh.at[dst_v.at[0]],
                     ssems.at[0], add=True)

    @pl.loop(1, NCHUNK - NBUF + 1, step=NBUF)
    def _(k0):
        for j in range(NBUF):
            k = k0 + j
            b = (1 + j) % NBUF
            rb = j % NBUF
            pltpu.make_async_copy(y_hbm.at[src_v.at[0]], rows.at[rb],
                                  ssems.at[rb]).wait()
            pltpu.async_copy(y_hbm.at[src_v.at[k + NBUF - 1]], rows.at[rb],
                             sems.at[rb])
            pltpu.make_async_copy(y_hbm.at[src_v.at[0]], rows.at[b],
                                  sems.at[b]).wait()
            pltpu.async_copy(rows.at[b], acc_sh.at[dst_v.at[k]],
                             ssems.at[b], add=True)

    for t in range(NBUF - 1):
        k = NCHUNK - NBUF + 1 + t
        b = (1 + t) % NBUF
        pltpu.make_async_copy(y_hbm.at[src_v.at[0]], rows.at[b],
                              sems.at[b]).wait()
        pltpu.async_copy(rows.at[b], acc_sh.at[dst_v.at[k]],
                         ssems.at[b], add=True)
    for b in range(NBUF):
        pltpu.make_async_copy(y_hbm.at[src_v.at[0]], rows.at[b],
                              ssems.at[b]).wait()

    plsc.subcore_barrier()

    pltpu.sync_copy(acc_sh.at[pl.ds(base, RQ)],
                    out_hbm.at[c, pl.ds(base, RQ)])

    @pl.when(s == NS - 1)
    def _():
        pltpu.sync_copy(acc_sh.at[pl.ds(NS * RQ, N - NS * RQ)],
                        out_hbm.at[c, pl.ds(NS * RQ, N - NS * RQ)])


def _sc_agg(y2, src2, dst):
    mesh = plsc.VectorSubcoreMesh(core_axis_name="c", subcore_axis_name="s",
                                  num_cores=NC, num_subcores=NS)
    f = pl.kernel(
        _sc_agg_body,
        out_type=jax.ShapeDtypeStruct((NC, N, F), jnp.float32),
        mesh=mesh,
        scratch_types=[
            pltpu.VMEM((NCHUNK, CH), jnp.int32),
            pltpu.VMEM((NCHUNK, CH), jnp.int32),
            pltpu.VMEM((NBUF, CH, F), jnp.float32),
            pltpu.VMEM((ZR, F), jnp.float32),
            pltpu.VMEM_SHARED((N + PADN, F), jnp.float32),
            pltpu.SemaphoreType.DMA((NBUF,)),
            pltpu.SemaphoreType.DMA((NBUF,)),
        ],
        compiler_params=pltpu.CompilerParams(use_tc_tiling_on_sc=False),
    )
    return f(y2, src2, dst)


def _tc_layer_body(p_ref, w_ref, b_ref, c_ref, o_ref, *, poly):
    x = jnp.concatenate([p_ref[0], p_ref[1]], axis=-1)
    h = jnp.dot(x, w_ref[...], preferred_element_type=jnp.float32) + b_ref[...]
    if poly:
        c0 = c_ref[0:1, :]
        c1 = c_ref[1:2, :]
        c2 = c_ref[2:3, :]
        c3 = c_ref[3:4, :]
        h = c0 + h * (c1 + h * (c2 + h * c3))
    o_ref[...] = h


def _tc_layer(partials, W, b, crow, poly):
    BN = 1000
    return pl.pallas_call(
        functools.partial(_tc_layer_body, poly=poly),
        grid=(N // BN,),
        in_specs=[
            pl.BlockSpec((NC, BN, F), lambda i: (0, i, 0)),
            pl.BlockSpec((D, D), lambda i: (0, 0)),
            pl.BlockSpec((1, D), lambda i: (0, 0)),
            pl.BlockSpec((ORDER + 1, D), lambda i: (0, 0)),
        ],
        out_specs=pl.BlockSpec((BN, D), lambda i: (i, 0)),
        out_shape=jax.ShapeDtypeStruct((N, D), jnp.float32),
    )(partials, W, b.reshape(1, D), crow)


def kernel(nf_mat, conv_mat, W1, b1, W2, b2, W3, b3, coeffs):
    pad = SLOTS - EPT
    pad_src = jnp.broadcast_to((jnp.arange(pad, dtype=jnp.int32) * 37) % N,
                               (NS, pad))
    pad_dst = jnp.broadcast_to(N + (jnp.arange(pad, dtype=jnp.int32) % PADN),
                               (NS, pad))
    src = jnp.concatenate([conv_mat[0].reshape(NS, EPT), pad_src], axis=1)
    dst = jnp.concatenate([conv_mat[1].reshape(NS, EPT), pad_dst], axis=1)
    src = src.reshape(NS, NCHUNK, CH)
    dst = dst.reshape(NS, NCHUNK, CH)
    src2 = jnp.stack([src * 2, src * 2 + 1])
    cb = jnp.broadcast_to(coeffs[:, :, None], (2, ORDER + 1, D))

    p = _sc_agg(nf_mat.reshape(NC * N, F), src2, dst)
    h = _tc_layer(p, W1, b1, cb[0], poly=True)
    p = _sc_agg(h.reshape(NC * N, F), src2, dst)
    h = _tc_layer(p, W2, b2, cb[1], poly=True)
    p = _sc_agg(h.reshape(NC * N, F), src2, dst)
    return _tc_layer(p, W3, b3, cb[0], poly=False)


# speedup vs baseline: 1.0940x; 1.0291x over previous
import functools

import jax
import jax.numpy as jnp
from jax import lax
from jax.experimental import pallas as pl
from jax.experimental.pallas import tpu as pltpu
from jax.experimental.pallas import tpu_sc as plsc

N = 10000
E = 320000
D = 128
ORDER = 3

NC = 2
NS = 16
F = D // NC
EPT = E // NS
CH = 125
NCHUNK = EPT // CH
NBUF = 5
RQ = 624
ZR = 104
LANES = 16


def _sc_agg_body(y_hbm, src_hbm, dst_hbm, out_hbm,
                 src_v, dst_v, rows, zbuf, acc_sh, sems, ssems):
    c = lax.axis_index("c")
    s = lax.axis_index("s")
    base = pl.multiple_of(s * RQ, 8)

    pltpu.sync_copy(src_hbm.at[c, s], src_v)
    pltpu.sync_copy(dst_hbm.at[s], dst_v)

    def zrow(i, carry):
        for j in range(F // LANES):
            zbuf[i, pl.ds(j * LANES, LANES)] = jnp.zeros((LANES,), jnp.float32)
        return carry
    lax.fori_loop(0, ZR, zrow, 0)
    for r in range(RQ // ZR):
        pltpu.sync_copy(zbuf, acc_sh.at[pl.ds(base + r * ZR, ZR)])

    @pl.when(s == NS - 1)
    def _():
        pltpu.sync_copy(zbuf.at[pl.ds(0, N - NS * RQ)],
                        acc_sh.at[pl.ds(NS * RQ, N - NS * RQ)])

    plsc.subcore_barrier()

    for b in range(NBUF):
        pltpu.async_copy(y_hbm.at[src_v.at[b]], rows.at[b], sems.at[b])
    pltpu.make_async_copy(y_hbm.at[src_v.at[0]], rows.at[0],
                          sems.at[0]).wait()
    pltpu.async_copy(rows.at[0], acc_sh.at[dst_v.at[0]],
                     ssems.at[0], add=True)

    @pl.loop(1, NCHUNK - NBUF + 1, step=NBUF)
    def _(k0):
        for j in range(NBUF):
            k = k0 + j
            b = (1 + j) % NBUF
            rb = j % NBUF
            pltpu.make_async_copy(y_hbm.at[src_v.at[0]], rows.at[rb],
                                  ssems.at[rb]).wait()
            pltpu.async_copy(y_hbm.at[src_v.at[k + NBUF - 1]], rows.at[rb],
                             sems.at[rb])
            pltpu.make_async_copy(y_hbm.at[src_v.at[0]], rows.at[b],
                                  sems.at[b]).wait()
            pltpu.async_copy(rows.at[b], acc_sh.at[dst_v.at[k]],
                             ssems.at[b], add=True)

    for t in range(NBUF - 1):
        k = NCHUNK - NBUF + 1 + t
        b = (1 + t) % NBUF
        pltpu.make_async_copy(y_hbm.at[src_v.at[0]], rows.at[b],
                              sems.at[b]).wait()
        pltpu.async_copy(rows.at[b], acc_sh.at[dst_v.at[k]],
                         ssems.at[b], add=True)
    for b in range(NBUF):
        pltpu.make_async_copy(y_hbm.at[src_v.at[0]], rows.at[b],
                              ssems.at[b]).wait()

    plsc.subcore_barrier()

    pltpu.sync_copy(acc_sh.at[pl.ds(base, RQ)],
                    out_hbm.at[c, pl.ds(base, RQ)])

    @pl.when(s == NS - 1)
    def _():
        pltpu.sync_copy(acc_sh.at[pl.ds(NS * RQ, N - NS * RQ)],
                        out_hbm.at[c, pl.ds(NS * RQ, N - NS * RQ)])


def _sc_agg(y2, src2, dst):
    mesh = plsc.VectorSubcoreMesh(core_axis_name="c", subcore_axis_name="s",
                                  num_cores=NC, num_subcores=NS)
    f = pl.kernel(
        _sc_agg_body,
        out_type=jax.ShapeDtypeStruct((NC, N, F), jnp.float32),
        mesh=mesh,
        scratch_types=[
            pltpu.VMEM((NCHUNK, CH), jnp.int32),
            pltpu.VMEM((NCHUNK, CH), jnp.int32),
            pltpu.VMEM((NBUF, CH, F), jnp.float32),
            pltpu.VMEM((ZR, F), jnp.float32),
            pltpu.VMEM_SHARED((N, F), jnp.float32),
            pltpu.SemaphoreType.DMA((NBUF,)),
            pltpu.SemaphoreType.DMA((NBUF,)),
        ],
        compiler_params=pltpu.CompilerParams(use_tc_tiling_on_sc=False),
    )
    return f(y2, src2, dst)


def _tc_layer_body(p_ref, w_ref, b_ref, c_ref, o_ref, *, poly):
    x = jnp.concatenate([p_ref[0], p_ref[1]], axis=-1)
    h = jnp.dot(x, w_ref[...], preferred_element_type=jnp.float32) + b_ref[...]
    if poly:
        c0 = c_ref[0:1, :]
        c1 = c_ref[1:2, :]
        c2 = c_ref[2:3, :]
        c3 = c_ref[3:4, :]
        h = c0 + h * (c1 + h * (c2 + h * c3))
    o_ref[...] = h


def _tc_layer(partials, W, b, crow, poly):
    BN = 1000
    return pl.pallas_call(
        functools.partial(_tc_layer_body, poly=poly),
        grid=(N // BN,),
        in_specs=[
            pl.BlockSpec((NC, BN, F), lambda i: (0, i, 0)),
            pl.BlockSpec((D, D), lambda i: (0, 0)),
            pl.BlockSpec((1, D), lambda i: (0, 0)),
            pl.BlockSpec((ORDER + 1, D), lambda i: (0, 0)),
        ],
        out_specs=pl.BlockSpec((BN, D), lambda i: (i, 0)),
        out_shape=jax.ShapeDtypeStruct((N, D), jnp.float32),
    )(partials, W, b.reshape(1, D), crow)


def kernel(nf_mat, conv_mat, W1, b1, W2, b2, W3, b3, coeffs):
    src = conv_mat[0].reshape(NS, NCHUNK, CH)
    dst = conv_mat[1].reshape(NS, NCHUNK, CH)
    src2 = jnp.stack([src * 2, src * 2 + 1])
    cb = jnp.broadcast_to(coeffs[:, :, None], (2, ORDER + 1, D))

    p = _sc_agg(nf_mat.reshape(NC * N, F), src2, dst)
    h = _tc_layer(p, W1, b1, cb[0], poly=True)
    p = _sc_agg(h.reshape(NC * N, F), src2, dst)
    h = _tc_layer(p, W2, b2, cb[1], poly=True)
    p = _sc_agg(h.reshape(NC * N, F), src2, dst)
    return _tc_layer(p, W3, b3, cb[0], poly=False)


# speedup vs baseline: 1.1217x; 1.0253x over previous
import functools

import jax
import jax.numpy as jnp
from jax import lax
from jax.experimental import pallas as pl
from jax.experimental.pallas import tpu as pltpu
from jax.experimental.pallas import tpu_sc as plsc

N = 10000
E = 320000
D = 128
ORDER = 3

NC = 2
NS = 16
F = D // NC
EPT = E // NS
CH = 125
NCHUNK = EPT // CH
NBUF = 5
RQ = 624
ZR = 104
LANES = 16


def _sc_agg_body(y_hbm, src_hbm, dst_hbm, out_hbm,
                 src_v, dst_v, rows, zbuf, acc_sh, sems, ssems, isems):
    c = lax.axis_index("c")
    s = lax.axis_index("s")
    base = pl.multiple_of(s * RQ, 8)

    pltpu.async_copy(src_hbm.at[c, s], src_v, isems.at[0])
    pltpu.async_copy(dst_hbm.at[s], dst_v, isems.at[1])

    def zrow(i, carry):
        for j in range(F // LANES):
            zbuf[i, pl.ds(j * LANES, LANES)] = jnp.zeros((LANES,), jnp.float32)
        return carry
    lax.fori_loop(0, ZR, zrow, 0)
    for r in range(RQ // ZR):
        pltpu.sync_copy(zbuf, acc_sh.at[pl.ds(base + r * ZR, ZR)])

    @pl.when(s == NS - 1)
    def _():
        pltpu.sync_copy(zbuf.at[pl.ds(0, N - NS * RQ)],
                        acc_sh.at[pl.ds(NS * RQ, N - NS * RQ)])

    pltpu.make_async_copy(src_hbm.at[c, s], src_v, isems.at[0]).wait()
    pltpu.make_async_copy(dst_hbm.at[s], dst_v, isems.at[1]).wait()

    plsc.subcore_barrier()

    for b in range(NBUF):
        pltpu.async_copy(y_hbm.at[src_v.at[b]], rows.at[b], sems.at[b])
    pltpu.make_async_copy(y_hbm.at[src_v.at[0]], rows.at[0],
                          sems.at[0]).wait()
    pltpu.async_copy(rows.at[0], acc_sh.at[dst_v.at[0]],
                     ssems.at[0], add=True)

    @pl.loop(1, NCHUNK - NBUF + 1, step=NBUF)
    def _(k0):
        for j in range(NBUF):
            k = k0 + j
            b = (1 + j) % NBUF
            rb = j % NBUF
            pltpu.make_async_copy(y_hbm.at[src_v.at[0]], rows.at[rb],
                                  ssems.at[rb]).wait()
            pltpu.async_copy(y_hbm.at[src_v.at[k + NBUF - 1]], rows.at[rb],
                             sems.at[rb])
            pltpu.make_async_copy(y_hbm.at[src_v.at[0]], rows.at[b],
                                  sems.at[b]).wait()
            pltpu.async_copy(rows.at[b], acc_sh.at[dst_v.at[k]],
                             ssems.at[b], add=True)

    for t in range(NBUF - 1):
        k = NCHUNK - NBUF + 1 + t
        b = (1 + t) % NBUF
        pltpu.make_async_copy(y_hbm.at[src_v.at[0]], rows.at[b],
                              sems.at[b]).wait()
        pltpu.async_copy(rows.at[b], acc_sh.at[dst_v.at[k]],
                         ssems.at[b], add=True)
    for b in range(NBUF):
        pltpu.make_async_copy(y_hbm.at[src_v.at[0]], rows.at[b],
                              ssems.at[b]).wait()

    plsc.subcore_barrier()

    pltpu.sync_copy(acc_sh.at[pl.ds(base, RQ)],
                    out_hbm.at[c, pl.ds(base, RQ)])

    @pl.when(s == NS - 1)
    def _():
        pltpu.sync_copy(acc_sh.at[pl.ds(NS * RQ, N - NS * RQ)],
                        out_hbm.at[c, pl.ds(NS * RQ, N - NS * RQ)])


def _sc_agg(y2, src2, dst):
    mesh = plsc.VectorSubcoreMesh(core_axis_name="c", subcore_axis_name="s",
                                  num_cores=NC, num_subcores=NS)
    f = pl.kernel(
        _sc_agg_body,
        out_type=jax.ShapeDtypeStruct((NC, N, F), jnp.float32),
        mesh=mesh,
        scratch_types=[
            pltpu.VMEM((NCHUNK, CH), jnp.int32),
            pltpu.VMEM((NCHUNK, CH), jnp.int32),
            pltpu.VMEM((NBUF, CH, F), jnp.float32),
            pltpu.VMEM((ZR, F), jnp.float32),
            pltpu.VMEM_SHARED((N, F), jnp.float32),
            pltpu.SemaphoreType.DMA((NBUF,)),
            pltpu.SemaphoreType.DMA((NBUF,)),
            pltpu.SemaphoreType.DMA((2,)),
        ],
        compiler_params=pltpu.CompilerParams(use_tc_tiling_on_sc=False),
    )
    return f(y2, src2, dst)


def _tc_layer_body(p_ref, w_ref, b_ref, c_ref, o_ref, *, poly):
    x = jnp.concatenate([p_ref[0], p_ref[1]], axis=-1)
    h = jnp.dot(x, w_ref[...], preferred_element_type=jnp.float32) + b_ref[...]
    if poly:
        c0 = c_ref[0:1, :]
        c1 = c_ref[1:2, :]
        c2 = c_ref[2:3, :]
        c3 = c_ref[3:4, :]
        h = c0 + h * (c1 + h * (c2 + h * c3))
    o_ref[...] = h


def _tc_layer(partials, W, b, crow, poly):
    BN = 1000
    return pl.pallas_call(
        functools.partial(_tc_layer_body, poly=poly),
        grid=(N // BN,),
        in_specs=[
            pl.BlockSpec((NC, BN, F), lambda i: (0, i, 0)),
            pl.BlockSpec((D, D), lambda i: (0, 0)),
            pl.BlockSpec((1, D), lambda i: (0, 0)),
            pl.BlockSpec((ORDER + 1, D), lambda i: (0, 0)),
        ],
        out_specs=pl.BlockSpec((BN, D), lambda i: (i, 0)),
        out_shape=jax.ShapeDtypeStruct((N, D), jnp.float32),
    )(partials, W, b.reshape(1, D), crow)


def kernel(nf_mat, conv_mat, W1, b1, W2, b2, W3, b3, coeffs):
    src = conv_mat[0].reshape(NS, NCHUNK, CH)
    dst = conv_mat[1].reshape(NS, NCHUNK, CH)
    src2 = jnp.stack([src * 2, src * 2 + 1])
    cb = jnp.broadcast_to(coeffs[:, :, None], (2, ORDER + 1, D))

    p = _sc_agg(nf_mat.reshape(NC * N, F), src2, dst)
    h = _tc_layer(p, W1, b1, cb[0], poly=True)
    p = _sc_agg(h.reshape(NC * N, F), src2, dst)
    h = _tc_layer(p, W2, b2, cb[1], poly=True)
    p = _sc_agg(h.reshape(NC * N, F), src2, dst)
    return _tc_layer(p, W3, b3, cb[0], poly=False)
